# baseline (speedup 1.0000x reference)
import functools

import jax
import jax.numpy as jnp
from jax import lax
from jax.experimental import pallas as pl
from jax.experimental.pallas import tpu as pltpu
from jax.experimental.pallas import tpu_sc as plsc

_NBUF = 4


def _make_gather(V, D, B):
    info = plsc.get_sparse_core_info()
    nc, ns = info.num_cores, info.num_subcores
    nw = nc * ns
    assert B % (8 * nw * _NBUF) == 0 and D % info.num_lanes == 0
    b_per_w = B // nw
    rows_c = b_per_w // _NBUF
    mesh = plsc.VectorSubcoreMesh(core_axis_name="c", subcore_axis_name="s")

    @functools.partial(
        pl.kernel,
        mesh=mesh,
        out_type=jax.ShapeDtypeStruct((B, D), jnp.float32),
        scratch_types=[
            pltpu.VMEM((_NBUF, rows_c), jnp.int32),
            pltpu.VMEM((_NBUF, rows_c, D), jnp.float32),
            pltpu.SemaphoreType.DMA((_NBUF,)),
            pltpu.SemaphoreType.DMA,
        ],
    )
    def gather_kernel(idx_hbm, table_hbm, out_hbm, idx_v, rows_v, gsems, wsem):
        wid = lax.axis_index("s") * nc + lax.axis_index("c")
        base = wid * b_per_w
        pltpu.sync_copy(idx_hbm.at[wid], idx_v)
        gathers = [
            pltpu.async_copy(table_hbm.at[idx_v.at[c]], rows_v.at[c], gsems.at[c])
            for c in range(_NBUF)
        ]
        writes = []
        for c in range(_NBUF):
            gathers[c].wait()
            writes.append(
                pltpu.async_copy(
                    rows_v.at[c], out_hbm.at[pl.ds(base + c * rows_c, rows_c)], wsem
                )
            )
        for w in writes:
            w.wait()

    return gather_kernel


def kernel(class_ids, prototypes):
    V, D = prototypes.shape
    (B,) = class_ids.shape
    info = plsc.get_sparse_core_info()
    nw = info.num_cores * info.num_subcores
    gather = _make_gather(V, D, B)
    idx = class_ids.astype(jnp.int32).reshape(nw, _NBUF, B // (nw * _NBUF))
    return gather(idx, prototypes)


# speedup vs baseline: 1.0063x; 1.0063x over previous
import functools

import jax
import jax.numpy as jnp
from jax import lax
from jax.experimental import pallas as pl
from jax.experimental.pallas import tpu as pltpu
from jax.experimental.pallas import tpu_sc as plsc

_NBUF = 2


def _make_gather(V, D, B):
    info = plsc.get_sparse_core_info()
    nc, ns = info.num_cores, info.num_subcores
    nw = nc * ns
    assert B % (8 * nw * _NBUF) == 0 and D % info.num_lanes == 0
    b_per_w = B // nw
    rows_c = b_per_w // _NBUF
    mesh = plsc.VectorSubcoreMesh(core_axis_name="c", subcore_axis_name="s")

    @functools.partial(
        pl.kernel,
        mesh=mesh,
        out_type=jax.ShapeDtypeStruct((B, D), jnp.float32),
        scratch_types=[
            pltpu.VMEM((_NBUF, rows_c), jnp.int32),
            pltpu.VMEM((_NBUF, rows_c, D), jnp.float32),
            pltpu.SemaphoreType.DMA((_NBUF,)),
            pltpu.SemaphoreType.DMA,
        ],
    )
    def gather_kernel(idx_hbm, table_hbm, out_hbm, idx_v, rows_v, gsems, wsem):
        wid = lax.axis_index("s") * nc + lax.axis_index("c")
        base = wid * b_per_w
        pltpu.sync_copy(idx_hbm.at[wid], idx_v)
        gathers = [
            pltpu.async_copy(table_hbm.at[idx_v.at[c]], rows_v.at[c], gsems.at[c])
            for c in range(_NBUF)
        ]
        writes = []
        for c in range(_NBUF):
            gathers[c].wait()
            writes.append(
                pltpu.async_copy(
                    rows_v.at[c], out_hbm.at[pl.ds(base + c * rows_c, rows_c)], wsem
                )
            )
        for w in writes:
            w.wait()

    return gather_kernel


def kernel(class_ids, prototypes):
    V, D = prototypes.shape
    (B,) = class_ids.shape
    info = plsc.get_sparse_core_info()
    nw = info.num_cores * info.num_subcores
    gather = _make_gather(V, D, B)
    idx = class_ids.astype(jnp.int32).reshape(nw, _NBUF, B // (nw * _NBUF))
    return gather(idx, prototypes)


# speedup vs baseline: 1.0134x; 1.0071x over previous
import functools

import jax
import jax.numpy as jnp
from jax import lax
from jax.experimental import pallas as pl
from jax.experimental.pallas import tpu as pltpu
from jax.experimental.pallas import tpu_sc as plsc


def _make_gather(V, D, B):
    info = plsc.get_sparse_core_info()
    nc, ns = info.num_cores, info.num_subcores
    nw = nc * ns
    assert B % (8 * nw) == 0 and D % info.num_lanes == 0
    b_per_w = B // nw
    mesh = plsc.VectorSubcoreMesh(core_axis_name="c", subcore_axis_name="s")

    @functools.partial(
        pl.kernel,
        mesh=mesh,
        out_type=jax.ShapeDtypeStruct((B, D), jnp.float32),
        scratch_types=[
            pltpu.VMEM((b_per_w,), jnp.int32),
            pltpu.VMEM((b_per_w, D), jnp.float32),
            pltpu.SemaphoreType.DMA,
        ],
    )
    def gather_kernel(idx_hbm, table_hbm, out_hbm, idx_v, rows_v, sem):
        wid = lax.axis_index("s") * nc + lax.axis_index("c")
        base = wid * b_per_w
        pltpu.sync_copy(idx_hbm.at[pl.ds(base, b_per_w)], idx_v)
        pltpu.async_copy(table_hbm.at[idx_v], rows_v, sem).wait()
        pltpu.sync_copy(rows_v, out_hbm.at[pl.ds(base, b_per_w)])

    return gather_kernel


def kernel(class_ids, prototypes):
    V, D = prototypes.shape
    (B,) = class_ids.shape
    gather = _make_gather(V, D, B)
    return gather(class_ids.astype(jnp.int32), prototypes)
